# baseline (speedup 1.0000x reference)
import functools

import jax
import jax.numpy as jnp
from jax import lax
from jax.experimental import pallas as pl
from jax.experimental.pallas import tpu as pltpu
from jax.experimental.pallas import tpu_sc as plsc

_N = 1024
_IN = 128
_H = 64
_E = 32768

_NC = 2
_NS = 16
_EPT = _E // _NS
_HALF = (_N * _N) // _NC
_WPT = _HALF // _NS
_BUF = 16384
_NCHUNK = _EPT // 128


def _sc_body(edges_hbm, out_hbm, src_v, dst_v, idx_refs, ones_v, buf_v,
             C_sh, sem):
    cid = lax.axis_index("c")
    sid = lax.axis_index("s")

    def zstep(i, _):
        for j in range(8):
            buf_v[pl.ds(i * 128 + j * 16, 16)] = jnp.zeros((16,), jnp.float32)
        return 0
    lax.fori_loop(0, _BUF // 128, zstep, 0)
    zs = [
        pltpu.async_copy(buf_v, C_sh.at[pl.ds(sid * _WPT + k * _BUF, _BUF)], sem)
        for k in range(_WPT // _BUF)
    ]

    base = sid * _EPT
    pltpu.sync_copy(edges_hbm.at[0, pl.ds(base, _EPT)], src_v)
    pltpu.sync_copy(edges_hbm.at[1, pl.ds(base, _EPT)], dst_v)

    cbase = cid * _HALF
    for c in range(_NCHUNK):
        for j in range(8):
            o = c * 128 + j * 16
            s = src_v[pl.ds(o, 16)]
            d = dst_v[pl.ds(o, 16)]
            off = (((d >> 3) << 13) | ((s >> 7) << 10)
                   | ((d & 7) << 7) | (s & 127)) - cbase
            off = jnp.where((d >> 9) == cid, off, _HALF)
            idx_refs[c][pl.ds(j * 16, 16)] = off
    for j in range(8):
        ones_v[pl.ds(j * 16, 16)] = jnp.full((16,), 1.0, jnp.float32)

    for z in zs:
        z.wait()
    plsc.subcore_barrier()

    hs = [
        pltpu.async_copy(ones_v, C_sh.at[idx_refs[c]], sem, add=True)
        for c in range(_NCHUNK)
    ]
    for h in hs:
        h.wait()
    plsc.subcore_barrier()

    pltpu.sync_copy(
        C_sh.at[pl.ds(sid * _WPT, _WPT)],
        out_hbm.at[pl.ds(cid * _HALF + sid * _WPT, _WPT)],
    )


_sc_call = pl.kernel(
    _sc_body,
    out_type=jax.ShapeDtypeStruct((_N * _N,), jnp.float32),
    mesh=plsc.VectorSubcoreMesh(core_axis_name="c", subcore_axis_name="s"),
    scratch_types=[
        pltpu.VMEM((_EPT,), jnp.int32),
        pltpu.VMEM((_EPT,), jnp.int32),
        [pltpu.VMEM((128,), jnp.int32) for _ in range(_NCHUNK)],
        pltpu.VMEM((128,), jnp.float32),
        pltpu.VMEM((_BUF,), jnp.float32),
        pltpu.VMEM_SHARED((_HALF + 128,), jnp.float32),
        pltpu.SemaphoreType.DMA,
    ],
)


def _dense_body(Cb_ref, x_ref, W1_ref, b1_ref, W2_ref, b2_ref, Wo_ref,
                bo_ref, We_ref, be_ref, out_ref):
    f32 = jnp.float32

    def cslice(c2):
        return Cb_ref[:, c2].reshape(_N, 128)

    deg = jnp.ones((_N, 1), f32)
    for c2 in range(8):
        deg = deg + jnp.sum(cslice(c2), axis=1, keepdims=True)
    dinv = lax.rsqrt(deg)

    def conv(h, W, b):
        t = dinv * jnp.dot(h, W, preferred_element_type=f32)
        agg = t
        for c2 in range(8):
            agg = agg + jnp.dot(cslice(c2), t[c2 * 128:(c2 + 1) * 128, :],
                                preferred_element_type=f32)
        return jnp.maximum(dinv * agg + b, 0.0)

    h = conv(x_ref[...], W1_ref[...], b1_ref[...])
    h = conv(h, W2_ref[...], b2_ref[...])

    We_top = We_ref[: _H, :]
    We_bot = We_ref[_H:, :]
    u1 = jnp.dot(Wo_ref[...], We_top, preferred_element_type=f32)
    u2 = jnp.dot(Wo_ref[...], We_bot, preferred_element_type=f32)
    a = jnp.dot(h, u1, preferred_element_type=f32)
    a = a + jnp.dot(bo_ref[...], We_top, preferred_element_type=f32)
    b_row = lax.dot_general(u2, h, (((0,), (1,)), ((), ())),
                            preferred_element_type=f32)
    b_row = b_row + jnp.dot(bo_ref[...], We_bot, preferred_element_type=f32)
    out_ref[...] = a + b_row + be_ref[...]


_dense_call = pl.pallas_call(
    _dense_body,
    out_shape=jax.ShapeDtypeStruct((_N, _N), jnp.float32),
)


def kernel(x, edge_index, W1, b1, W2, b2, Wo, bo, We, be):
    Cb = _sc_call(edge_index).reshape(_N // 8, 8, 8, 128)
    return _dense_call(
        Cb, x, W1,
        b1.reshape(1, _H), W2, b2.reshape(1, _H), Wo,
        bo.reshape(1, _H), We, be.reshape(1, 1),
    )


# speedup vs baseline: 1.4852x; 1.4852x over previous
import functools

import jax
import jax.numpy as jnp
from jax import lax
from jax.experimental import pallas as pl
from jax.experimental.pallas import tpu as pltpu
from jax.experimental.pallas import tpu_sc as plsc

_N = 1024
_IN = 128
_H = 64
_E = 32768

_NC = 2
_NS = 16
_EPT = _E // _NS
_HALF = (_N * _N) // _NC
_WPT = _HALF // _NS
_BUF = 16384
_NCHUNK = _EPT // 128


def _sc_body(edges_hbm, out_hbm, src_v, dst_v, idx_refs, ones_v, buf_v,
             C_sh, sem):
    cid = lax.axis_index("c")
    sid = lax.axis_index("s")

    def zstep(i, _):
        for j in range(8):
            buf_v[pl.ds(i * 128 + j * 16, 16)] = jnp.zeros((16,), jnp.float32)
        return 0
    lax.fori_loop(0, _BUF // 128, zstep, 0)
    zs = [
        pltpu.async_copy(buf_v, C_sh.at[pl.ds(sid * _WPT + k * _BUF, _BUF)], sem)
        for k in range(_WPT // _BUF)
    ]

    base = sid * _EPT
    pltpu.sync_copy(edges_hbm.at[0, pl.ds(base, _EPT)], src_v)
    pltpu.sync_copy(edges_hbm.at[1, pl.ds(base, _EPT)], dst_v)

    cbase = cid * _HALF
    for c in range(_NCHUNK):
        for j in range(8):
            o = c * 128 + j * 16
            s = src_v[pl.ds(o, 16)]
            d = dst_v[pl.ds(o, 16)]
            off = (((d >> 3) << 13) | ((s >> 7) << 10)
                   | ((d & 7) << 7) | (s & 127)) - cbase
            off = jnp.where((d >> 9) == cid, off, _HALF + (s & 127))
            idx_refs[c][pl.ds(j * 16, 16)] = off
    for j in range(8):
        ones_v[pl.ds(j * 16, 16)] = jnp.full((16,), 1.0, jnp.float32)

    for z in zs:
        z.wait()
    plsc.subcore_barrier()

    hs = [
        pltpu.async_copy(ones_v, C_sh.at[idx_refs[c]], sem, add=True)
        for c in range(_NCHUNK)
    ]
    for h in hs:
        h.wait()
    plsc.subcore_barrier()

    pltpu.sync_copy(
        C_sh.at[pl.ds(sid * _WPT, _WPT)],
        out_hbm.at[pl.ds(cid * _HALF + sid * _WPT, _WPT)],
    )


_sc_call = pl.kernel(
    _sc_body,
    out_type=jax.ShapeDtypeStruct((_N * _N,), jnp.float32),
    mesh=plsc.VectorSubcoreMesh(core_axis_name="c", subcore_axis_name="s"),
    scratch_types=[
        pltpu.VMEM((_EPT,), jnp.int32),
        pltpu.VMEM((_EPT,), jnp.int32),
        [pltpu.VMEM((128,), jnp.int32) for _ in range(_NCHUNK)],
        pltpu.VMEM((128,), jnp.float32),
        pltpu.VMEM((_BUF,), jnp.float32),
        pltpu.VMEM_SHARED((_HALF + 128,), jnp.float32),
        pltpu.SemaphoreType.DMA,
    ],
)


def _dense_body(Cb_ref, x_ref, W1_ref, b1_ref, W2_ref, b2_ref, Wo_ref,
                bo_ref, We_ref, be_ref, out_ref):
    f32 = jnp.float32

    def cslice(c2):
        return Cb_ref[:, c2].reshape(_N, 128)

    deg = jnp.ones((_N, 1), f32)
    for c2 in range(8):
        deg = deg + jnp.sum(cslice(c2), axis=1, keepdims=True)
    dinv = lax.rsqrt(deg)

    def conv(h, W, b):
        t = dinv * jnp.dot(h, W, preferred_element_type=f32)
        agg = t
        for c2 in range(8):
            agg = agg + jnp.dot(cslice(c2), t[c2 * 128:(c2 + 1) * 128, :],
                                preferred_element_type=f32)
        return jnp.maximum(dinv * agg + b, 0.0)

    h = conv(x_ref[...], W1_ref[...], b1_ref[...])
    h = conv(h, W2_ref[...], b2_ref[...])

    We_top = We_ref[: _H, :]
    We_bot = We_ref[_H:, :]
    u1 = jnp.dot(Wo_ref[...], We_top, preferred_element_type=f32)
    u2 = jnp.dot(Wo_ref[...], We_bot, preferred_element_type=f32)
    a = jnp.dot(h, u1, preferred_element_type=f32)
    a = a + jnp.dot(bo_ref[...], We_top, preferred_element_type=f32)
    b_row = lax.dot_general(u2, h, (((0,), (1,)), ((), ())),
                            preferred_element_type=f32)
    b_row = b_row + jnp.dot(bo_ref[...], We_bot, preferred_element_type=f32)
    out_ref[...] = a + b_row + be_ref[...]


_dense_call = pl.pallas_call(
    _dense_body,
    out_shape=jax.ShapeDtypeStruct((_N, _N), jnp.float32),
)


def kernel(x, edge_index, W1, b1, W2, b2, Wo, bo, We, be):
    Cb = _sc_call(edge_index).reshape(_N // 8, 8, 8, 128)
    return _dense_call(
        Cb, x, W1,
        b1.reshape(1, _H), W2, b2.reshape(1, _H), Wo,
        bo.reshape(1, _H), We, be.reshape(1, 1),
    )
